# Initial kernel scaffold; baseline (speedup 1.0000x reference)
import functools

import jax
import jax.numpy as jnp
from jax import lax
from jax.experimental import pallas as pl
from jax.experimental.pallas import tpu as pltpu
from jax.experimental.pallas import tpu_sc as plsc

NC, NS, LANES = 2, 16, 16
TILES = NC * NS
K = 128


def _edge_proj_body(ea_ref, w_ref, b_ref, o_ref):
    o_ref[...] = (
        jnp.dot(ea_ref[...], w_ref[...], preferred_element_type=jnp.float32)
        + b_ref[...]
    )


def _edge_proj(ea, w_t, b_row):
    Ep, DE = ea.shape
    D = w_t.shape[1]
    BE = 2048
    assert Ep % BE == 0
    return pl.pallas_call(
        _edge_proj_body,
        grid=(Ep // BE,),
        in_specs=[
            pl.BlockSpec((BE, DE), lambda i: (i, 0)),
            pl.BlockSpec((DE, D), lambda i: (0, 0)),
            pl.BlockSpec((1, D), lambda i: (0, 0)),
        ],
        out_specs=pl.BlockSpec((BE, D), lambda i: (i, 0)),
        out_shape=jax.ShapeDtypeStruct((Ep, D), jnp.float32),
        compiler_params=pltpu.CompilerParams(
            dimension_semantics=("parallel",)
        ),
    )(ea, w_t, b_row)


def _sc_agg(x, src, dst, e, zblk, n_chunks, n_pad):
    N, D = x.shape

    def body(x_hbm, src_hbm, dst_hbm, e_hbm, z_hbm, out_hbm,
             idx_s, idx_d, xj, ev, agg_sh):
        cid = lax.axis_index("c")
        sid = lax.axis_index("s")
        wid = sid * NC + cid
        zrows = n_pad // NS
        pltpu.sync_copy(z_hbm, agg_sh.at[pl.ds(sid * zrows, zrows)])
        plsc.subcore_barrier()

        base = wid * (n_chunks * K)

        @pl.loop(0, n_chunks)
        def _chunk(j):
            off = base + j * K
            pltpu.sync_copy(src_hbm.at[pl.ds(off, K)], idx_s)
            pltpu.sync_copy(dst_hbm.at[pl.ds(off, K)], idx_d)
            pltpu.sync_copy(x_hbm.at[idx_s], xj)
            pltpu.sync_copy(e_hbm.at[pl.ds(off, K), :], ev)
            for cc in range(D // LANES):
                col = pl.ds(cc * LANES, LANES)

                @pl.loop(0, K)
                def _row(r):
                    v = xj[r, col] + ev[r, col]
                    xj[r, col] = jnp.maximum(v, 0.0)

            pltpu.sync_copy(xj, agg_sh.at[idx_d], add=True)

        plsc.subcore_barrier()
        drows = N // NS
        pltpu.sync_copy(
            agg_sh.at[pl.ds(sid * drows, drows)],
            out_hbm.at[cid, pl.ds(sid * drows, drows), :],
        )

    mesh = plsc.VectorSubcoreMesh(core_axis_name="c", subcore_axis_name="s")
    kern = pl.kernel(
        body,
        out_type=jax.ShapeDtypeStruct((NC, N, D), jnp.float32),
        mesh=mesh,
        scratch_types=[
            pltpu.VMEM((K,), jnp.int32),
            pltpu.VMEM((K,), jnp.int32),
            pltpu.VMEM((K, D), jnp.float32),
            pltpu.VMEM((K, D), jnp.float32),
            pltpu.VMEM_SHARED((n_pad, D), jnp.float32),
        ],
    )
    return kern(x, src, dst, e, zblk)


def _node_mlp_body(x_ref, p_ref, w_ref, b_ref, o_ref):
    s = x_ref[...] + p_ref[0] + p_ref[1]
    h = jnp.dot(s, w_ref[...], preferred_element_type=jnp.float32) + b_ref[...]
    o_ref[...] = jnp.maximum(h, 0.0)


def _node_mlp(x, partials, w_t, b_row):
    N, D = x.shape
    BN = 1000
    assert N % BN == 0
    return pl.pallas_call(
        _node_mlp_body,
        grid=(N // BN,),
        in_specs=[
            pl.BlockSpec((BN, D), lambda i: (i, 0)),
            pl.BlockSpec((NC, BN, D), lambda i: (0, i, 0)),
            pl.BlockSpec((D, D), lambda i: (0, 0)),
            pl.BlockSpec((1, D), lambda i: (0, 0)),
        ],
        out_specs=pl.BlockSpec((BN, D), lambda i: (i, 0)),
        out_shape=jax.ShapeDtypeStruct((N, D), jnp.float32),
        compiler_params=pltpu.CompilerParams(
            dimension_semantics=("parallel",)
        ),
    )(x, partials, w_t, b_row)


def kernel(x, edge_index, edge_attr, W_edge, b_edge, W_mlp, b_mlp):
    N, D = x.shape
    E = edge_index.shape[1]
    DE = edge_attr.shape[1]

    per_round = TILES * K
    n_chunks = -(-E // per_round)
    Ep = per_round * n_chunks
    pad = Ep - E

    src = edge_index[0]
    dst = edge_index[1]
    ea = edge_attr
    if pad:
        src = jnp.concatenate([src, jnp.zeros((pad,), src.dtype)])
        dst = jnp.concatenate([dst, jnp.full((pad,), N, dst.dtype)])
        ea = jnp.concatenate([ea, jnp.zeros((pad, DE), ea.dtype)])

    n_pad = ((N + 1 + NS - 1) // NS) * NS
    zblk = jnp.zeros((n_pad // NS, D), jnp.float32)

    e = _edge_proj(ea, W_edge.T, b_edge[None, :])
    partials = _sc_agg(x, src, dst, e, zblk, n_chunks, n_pad)
    return _node_mlp(x, partials, W_mlp.T, b_mlp[None, :])


# speedup vs baseline: 1.5316x; 1.5316x over previous
import functools

import jax
import jax.numpy as jnp
from jax import lax
from jax.experimental import pallas as pl
from jax.experimental.pallas import tpu as pltpu
from jax.experimental.pallas import tpu_sc as plsc

NC, NS, LANES = 2, 16, 16
TILES = NC * NS
K = 128


def _edge_proj_body(ea_ref, w_ref, b_ref, o_ref):
    o_ref[...] = (
        jnp.dot(ea_ref[...], w_ref[...], preferred_element_type=jnp.float32)
        + b_ref[...]
    )


def _edge_proj(ea, w_t, b_row):
    Ep, DE = ea.shape
    D = w_t.shape[1]
    BE = 2048
    assert Ep % BE == 0
    return pl.pallas_call(
        _edge_proj_body,
        grid=(Ep // BE,),
        in_specs=[
            pl.BlockSpec((BE, DE), lambda i: (i, 0)),
            pl.BlockSpec((DE, D), lambda i: (0, 0)),
            pl.BlockSpec((1, D), lambda i: (0, 0)),
        ],
        out_specs=pl.BlockSpec((BE, D), lambda i: (i, 0)),
        out_shape=jax.ShapeDtypeStruct((Ep, D), jnp.float32),
        compiler_params=pltpu.CompilerParams(
            dimension_semantics=("parallel",)
        ),
    )(ea, w_t, b_row)


def _sc_agg(x, src, dst, e, zblk, n_chunks, n_pad):
    N, D = x.shape

    def body(x_hbm, src_hbm, dst_hbm, e_hbm, z_hbm, out_hbm,
             idx_s, idx_d, xj, ev, agg_sh):
        cid = lax.axis_index("c")
        sid = lax.axis_index("s")
        wid = sid * NC + cid
        zrows = n_pad // NS
        pltpu.sync_copy(z_hbm, agg_sh.at[pl.ds(sid * zrows, zrows)])
        plsc.subcore_barrier()

        base = wid * (n_chunks * K)

        @pl.loop(0, n_chunks)
        def _chunk(j):
            off = base + j * K
            pltpu.sync_copy(src_hbm.at[pl.ds(off, K)], idx_s)
            pltpu.sync_copy(dst_hbm.at[pl.ds(off, K)], idx_d)
            pltpu.sync_copy(x_hbm.at[idx_s], xj)
            pltpu.sync_copy(e_hbm.at[pl.ds(off, K), :], ev)
            for cc in range(D // LANES):
                col = pl.ds(cc * LANES, LANES)

                @pl.loop(0, K)
                def _row(r):
                    v = xj[r, col] + ev[r, col]
                    xj[r, col] = jnp.maximum(v, 0.0)

            pltpu.sync_copy(xj, agg_sh.at[idx_d], add=True)

        plsc.subcore_barrier()
        drows = n_pad // NS
        pltpu.sync_copy(
            agg_sh.at[pl.ds(sid * drows, drows)],
            out_hbm.at[cid, pl.ds(sid * drows, drows), :],
        )

    mesh = plsc.VectorSubcoreMesh(core_axis_name="c", subcore_axis_name="s")
    kern = pl.kernel(
        body,
        out_type=jax.ShapeDtypeStruct((NC, n_pad, D), jnp.float32),
        mesh=mesh,
        scratch_types=[
            pltpu.VMEM((K,), jnp.int32),
            pltpu.VMEM((K,), jnp.int32),
            pltpu.VMEM((K, D), jnp.float32),
            pltpu.VMEM((K, D), jnp.float32),
            pltpu.VMEM_SHARED((n_pad, D), jnp.float32),
        ],
    )
    return kern(x, src, dst, e, zblk)


def _node_mlp_body(x_ref, p_ref, w_ref, b_ref, o_ref):
    s = x_ref[...] + p_ref[0] + p_ref[1]
    h = jnp.dot(s, w_ref[...], preferred_element_type=jnp.float32) + b_ref[...]
    o_ref[...] = jnp.maximum(h, 0.0)


def _node_mlp(x, partials, w_t, b_row):
    N, D = x.shape
    BN = 1000
    assert N % BN == 0
    return pl.pallas_call(
        _node_mlp_body,
        grid=(N // BN,),
        in_specs=[
            pl.BlockSpec((BN, D), lambda i: (i, 0)),
            pl.BlockSpec((NC, BN, D), lambda i: (0, i, 0)),
            pl.BlockSpec((D, D), lambda i: (0, 0)),
            pl.BlockSpec((1, D), lambda i: (0, 0)),
        ],
        out_specs=pl.BlockSpec((BN, D), lambda i: (i, 0)),
        out_shape=jax.ShapeDtypeStruct((N, D), jnp.float32),
        compiler_params=pltpu.CompilerParams(
            dimension_semantics=("parallel",)
        ),
    )(x, partials, w_t, b_row)


def kernel(x, edge_index, edge_attr, W_edge, b_edge, W_mlp, b_mlp):
    N, D = x.shape
    E = edge_index.shape[1]
    DE = edge_attr.shape[1]

    per_round = TILES * K
    n_chunks = -(-E // per_round)
    Ep = per_round * n_chunks
    pad = Ep - E

    src = edge_index[0]
    dst = edge_index[1]
    ea = edge_attr
    if pad:
        src = jnp.concatenate([src, jnp.zeros((pad,), src.dtype)])
        dst = jnp.concatenate([dst, jnp.full((pad,), N, dst.dtype)])
        ea = jnp.concatenate([ea, jnp.zeros((pad, DE), ea.dtype)])

    n_pad = ((N + 1 + 8 * NS - 1) // (8 * NS)) * (8 * NS)
    zblk = jnp.zeros((n_pad // NS, D), jnp.float32)

    e = _edge_proj(ea, W_edge.T, b_edge[None, :])
    partials = _sc_agg(x, src, dst, e, zblk, n_chunks, n_pad)
    return _node_mlp(x, partials[:, :N], W_mlp.T, b_mlp[None, :])


# speedup vs baseline: 2.0663x; 1.3491x over previous
import functools

import jax
import jax.numpy as jnp
from jax import lax
from jax.experimental import pallas as pl
from jax.experimental.pallas import tpu as pltpu
from jax.experimental.pallas import tpu_sc as plsc

NC, NS, LANES = 2, 16, 16
TILES = NC * NS
K = 80


def _edge_proj_body(ea_ref, w_ref, b_ref, o_ref):
    o_ref[...] = (
        jnp.dot(ea_ref[...], w_ref[...], preferred_element_type=jnp.float32)
        + b_ref[...]
    )


def _edge_proj(ea, w_t, b_row):
    Ep, DE = ea.shape
    D = w_t.shape[1]
    BE = TILES * K
    assert Ep % BE == 0
    return pl.pallas_call(
        _edge_proj_body,
        grid=(Ep // BE,),
        in_specs=[
            pl.BlockSpec((BE, DE), lambda i: (i, 0)),
            pl.BlockSpec((DE, D), lambda i: (0, 0)),
            pl.BlockSpec((1, D), lambda i: (0, 0)),
        ],
        out_specs=pl.BlockSpec((BE, D), lambda i: (i, 0)),
        out_shape=jax.ShapeDtypeStruct((Ep, D), jnp.float32),
        compiler_params=pltpu.CompilerParams(
            dimension_semantics=("parallel",)
        ),
    )(ea, w_t, b_row)


def _sc_agg(x, src, dst, e, zblk, n_chunks, n_pad):
    N, D = x.shape
    assert n_chunks % 2 == 0

    def body(x_hbm, src_hbm, dst_hbm, e_hbm, z_hbm, out_hbm,
             idx_s0, idx_d0, xj0, ev0, idx_s1, idx_d1, xj1, ev1,
             sg0, se0, sg1, sg1b, agg_sh):
        cid = lax.axis_index("c")
        sid = lax.axis_index("s")
        wid = sid * NC + cid
        zrows = n_pad // NS
        pltpu.sync_copy(z_hbm, agg_sh.at[pl.ds(sid * zrows, zrows)])
        plsc.subcore_barrier()

        base = wid * (n_chunks * K)
        bufs = ((idx_s0, idx_d0, xj0, ev0, sg0, se0),
                (idx_s1, idx_d1, xj1, ev1, sg1, sg1b))

        def start(j, b):
            idx_s, idx_d, xj, ev, sg, se = bufs[b]
            off = base + j * K
            pltpu.sync_copy(src_hbm.at[pl.ds(off, K)], idx_s)
            pltpu.sync_copy(dst_hbm.at[pl.ds(off, K)], idx_d)
            pltpu.make_async_copy(x_hbm.at[idx_s], xj, sg).start()
            pltpu.make_async_copy(e_hbm.at[pl.ds(off, K), :], ev, se).start()

        def finish(j, b):
            idx_s, idx_d, xj, ev, sg, se = bufs[b]
            off = base + j * K
            pltpu.make_async_copy(x_hbm.at[idx_s], xj, sg).wait()
            pltpu.make_async_copy(e_hbm.at[pl.ds(off, K), :], ev, se).wait()
            for cc in range(D // LANES):
                col = pl.ds(cc * LANES, LANES)

                @pl.loop(0, K)
                def _row(r):
                    v = xj[r, col] + ev[r, col]
                    xj[r, col] = jnp.maximum(v, 0.0)

            pltpu.sync_copy(xj, agg_sh.at[idx_d], add=True)

        start(0, 0)

        @pl.loop(0, n_chunks, step=2)
        def _chunk(t):
            start(t + 1, 1)
            finish(t, 0)

            @pl.when(t + 2 < n_chunks)
            def _():
                start(t + 2, 0)

            finish(t + 1, 1)

        plsc.subcore_barrier()
        drows = n_pad // NS
        pltpu.sync_copy(
            agg_sh.at[pl.ds(sid * drows, drows)],
            out_hbm.at[cid, pl.ds(sid * drows, drows), :],
        )

    mesh = plsc.VectorSubcoreMesh(core_axis_name="c", subcore_axis_name="s")
    kern = pl.kernel(
        body,
        out_type=jax.ShapeDtypeStruct((NC, n_pad, D), jnp.float32),
        mesh=mesh,
        scratch_types=[
            pltpu.VMEM((K,), jnp.int32),
            pltpu.VMEM((K,), jnp.int32),
            pltpu.VMEM((K, D), jnp.float32),
            pltpu.VMEM((K, D), jnp.float32),
            pltpu.VMEM((K,), jnp.int32),
            pltpu.VMEM((K,), jnp.int32),
            pltpu.VMEM((K, D), jnp.float32),
            pltpu.VMEM((K, D), jnp.float32),
            pltpu.SemaphoreType.DMA,
            pltpu.SemaphoreType.DMA,
            pltpu.SemaphoreType.DMA,
            pltpu.SemaphoreType.DMA,
            pltpu.VMEM_SHARED((n_pad, D), jnp.float32),
        ],
    )
    return kern(x, src, dst, e, zblk)


def _node_mlp_body(x_ref, p_ref, w_ref, b_ref, o_ref):
    s = x_ref[...] + p_ref[0] + p_ref[1]
    h = jnp.dot(s, w_ref[...], preferred_element_type=jnp.float32) + b_ref[...]
    o_ref[...] = jnp.maximum(h, 0.0)


def _node_mlp(x, partials, w_t, b_row):
    N, D = x.shape
    BN = 1000
    assert N % BN == 0
    return pl.pallas_call(
        _node_mlp_body,
        grid=(N // BN,),
        in_specs=[
            pl.BlockSpec((BN, D), lambda i: (i, 0)),
            pl.BlockSpec((NC, BN, D), lambda i: (0, i, 0)),
            pl.BlockSpec((D, D), lambda i: (0, 0)),
            pl.BlockSpec((1, D), lambda i: (0, 0)),
        ],
        out_specs=pl.BlockSpec((BN, D), lambda i: (i, 0)),
        out_shape=jax.ShapeDtypeStruct((N, D), jnp.float32),
        compiler_params=pltpu.CompilerParams(
            dimension_semantics=("parallel",)
        ),
    )(x, partials, w_t, b_row)


def kernel(x, edge_index, edge_attr, W_edge, b_edge, W_mlp, b_mlp):
    N, D = x.shape
    E = edge_index.shape[1]
    DE = edge_attr.shape[1]

    per_round = TILES * K
    n_chunks = -(-E // per_round)
    n_chunks += n_chunks % 2
    Ep = per_round * n_chunks
    pad = Ep - E

    src = edge_index[0]
    dst = edge_index[1]
    ea = edge_attr
    if pad:
        src = jnp.concatenate([src, jnp.zeros((pad,), src.dtype)])
        dst = jnp.concatenate([dst, jnp.full((pad,), N, dst.dtype)])
        ea = jnp.concatenate([ea, jnp.zeros((pad, DE), ea.dtype)])

    n_pad = ((N + 1 + 8 * NS - 1) // (8 * NS)) * (8 * NS)
    zblk = jnp.zeros((n_pad // NS, D), jnp.float32)

    e = _edge_proj(ea, W_edge.T, b_edge[None, :])
    partials = _sc_agg(x, src, dst, e, zblk, n_chunks, n_pad)
    return _node_mlp(x, partials[:, :N], W_mlp.T, b_mlp[None, :])


# speedup vs baseline: 2.9250x; 1.4156x over previous
import functools

import jax
import jax.numpy as jnp
from jax import lax
from jax.experimental import pallas as pl
from jax.experimental.pallas import tpu as pltpu
from jax.experimental.pallas import tpu_sc as plsc

NC, NS, LANES = 2, 16, 16
TILES = NC * NS
K = 80


def _edge_proj_body(ea_ref, w_ref, b_ref, o_ref):
    o_ref[...] = (
        jnp.dot(ea_ref[...], w_ref[...], preferred_element_type=jnp.float32)
        + b_ref[...]
    )


def _edge_proj(ea, w_t, b_row):
    Ep, DE = ea.shape
    D = w_t.shape[1]
    BE = TILES * K
    assert Ep % BE == 0
    return pl.pallas_call(
        _edge_proj_body,
        grid=(Ep // BE,),
        in_specs=[
            pl.BlockSpec((BE, DE), lambda i: (i, 0)),
            pl.BlockSpec((DE, D), lambda i: (0, 0)),
            pl.BlockSpec((1, D), lambda i: (0, 0)),
        ],
        out_specs=pl.BlockSpec((BE, D), lambda i: (i, 0)),
        out_shape=jax.ShapeDtypeStruct((Ep, D), jnp.float32),
        compiler_params=pltpu.CompilerParams(
            dimension_semantics=("parallel",)
        ),
    )(ea, w_t, b_row)


def _sc_agg(x, src, dst, e, zblk, n_chunks, n_pad):
    N, D = x.shape
    assert n_chunks % 2 == 0

    def body(x_hbm, src_hbm, dst_hbm, e_hbm, z_hbm, out_hbm,
             idx_s0, idx_d0, xj0, ev0, idx_s1, idx_d1, xj1, ev1,
             sg0, se0, sg1, sg1b, agg_sh):
        cid = lax.axis_index("c")
        sid = lax.axis_index("s")
        wid = sid * NC + cid
        zrows = n_pad // NS
        pltpu.sync_copy(z_hbm, agg_sh.at[pl.ds(sid * zrows, zrows)])
        plsc.subcore_barrier()

        base = wid * (n_chunks * K)
        bufs = ((idx_s0, idx_d0, xj0, ev0, sg0, se0),
                (idx_s1, idx_d1, xj1, ev1, sg1, sg1b))

        def start(j, b):
            idx_s, idx_d, xj, ev, sg, se = bufs[b]
            off = base + j * K
            pltpu.sync_copy(src_hbm.at[pl.ds(off, K)], idx_s)
            pltpu.sync_copy(dst_hbm.at[pl.ds(off, K)], idx_d)
            pltpu.make_async_copy(x_hbm.at[idx_s], xj, sg).start()
            pltpu.make_async_copy(e_hbm.at[pl.ds(off, K), :], ev, se).start()

        def finish(j, b):
            idx_s, idx_d, xj, ev, sg, se = bufs[b]
            off = base + j * K
            pltpu.make_async_copy(x_hbm.at[idx_s], xj, sg).wait()
            pltpu.make_async_copy(e_hbm.at[pl.ds(off, K), :], ev, se).wait()

            @plsc.parallel_loop(0, K, unroll=2)
            def _row(r):
                for cc in range(D // LANES):
                    col = pl.ds(cc * LANES, LANES)
                    xj[r, col] = jnp.maximum(xj[r, col] + ev[r, col], 0.0)

            pltpu.sync_copy(xj, agg_sh.at[idx_d], add=True)

        start(0, 0)

        @pl.loop(0, n_chunks, step=2)
        def _chunk(t):
            start(t + 1, 1)
            finish(t, 0)

            @pl.when(t + 2 < n_chunks)
            def _():
                start(t + 2, 0)

            finish(t + 1, 1)

        plsc.subcore_barrier()
        drows = n_pad // NS
        pltpu.sync_copy(
            agg_sh.at[pl.ds(sid * drows, drows)],
            out_hbm.at[cid, pl.ds(sid * drows, drows), :],
        )

    mesh = plsc.VectorSubcoreMesh(core_axis_name="c", subcore_axis_name="s")
    kern = pl.kernel(
        body,
        out_type=jax.ShapeDtypeStruct((NC, n_pad, D), jnp.float32),
        mesh=mesh,
        scratch_types=[
            pltpu.VMEM((K,), jnp.int32),
            pltpu.VMEM((K,), jnp.int32),
            pltpu.VMEM((K, D), jnp.float32),
            pltpu.VMEM((K, D), jnp.float32),
            pltpu.VMEM((K,), jnp.int32),
            pltpu.VMEM((K,), jnp.int32),
            pltpu.VMEM((K, D), jnp.float32),
            pltpu.VMEM((K, D), jnp.float32),
            pltpu.SemaphoreType.DMA,
            pltpu.SemaphoreType.DMA,
            pltpu.SemaphoreType.DMA,
            pltpu.SemaphoreType.DMA,
            pltpu.VMEM_SHARED((n_pad, D), jnp.float32),
        ],
    )
    return kern(x, src, dst, e, zblk)


def _node_mlp_body(x_ref, p_ref, w_ref, b_ref, o_ref):
    s = x_ref[...] + p_ref[0] + p_ref[1]
    h = jnp.dot(s, w_ref[...], preferred_element_type=jnp.float32) + b_ref[...]
    o_ref[...] = jnp.maximum(h, 0.0)


def _node_mlp(x, partials, w_t, b_row):
    N, D = x.shape
    BN = 1000
    assert N % BN == 0
    return pl.pallas_call(
        _node_mlp_body,
        grid=(N // BN,),
        in_specs=[
            pl.BlockSpec((BN, D), lambda i: (i, 0)),
            pl.BlockSpec((NC, BN, D), lambda i: (0, i, 0)),
            pl.BlockSpec((D, D), lambda i: (0, 0)),
            pl.BlockSpec((1, D), lambda i: (0, 0)),
        ],
        out_specs=pl.BlockSpec((BN, D), lambda i: (i, 0)),
        out_shape=jax.ShapeDtypeStruct((N, D), jnp.float32),
        compiler_params=pltpu.CompilerParams(
            dimension_semantics=("parallel",)
        ),
    )(x, partials, w_t, b_row)


def kernel(x, edge_index, edge_attr, W_edge, b_edge, W_mlp, b_mlp):
    N, D = x.shape
    E = edge_index.shape[1]
    DE = edge_attr.shape[1]

    per_round = TILES * K
    n_chunks = -(-E // per_round)
    n_chunks += n_chunks % 2
    Ep = per_round * n_chunks
    pad = Ep - E

    src = edge_index[0]
    dst = edge_index[1]
    ea = edge_attr
    if pad:
        src = jnp.concatenate([src, jnp.zeros((pad,), src.dtype)])
        dst = jnp.concatenate([dst, jnp.full((pad,), N, dst.dtype)])
        ea = jnp.concatenate([ea, jnp.zeros((pad, DE), ea.dtype)])

    n_pad = ((N + 1 + 8 * NS - 1) // (8 * NS)) * (8 * NS)
    zblk = jnp.zeros((n_pad // NS, D), jnp.float32)

    e = _edge_proj(ea, W_edge.T, b_edge[None, :])
    partials = _sc_agg(x, src, dst, e, zblk, n_chunks, n_pad)
    return _node_mlp(x, partials[:, :N], W_mlp.T, b_mlp[None, :])


# speedup vs baseline: 2.9572x; 1.0110x over previous
import functools

import jax
import jax.numpy as jnp
from jax import lax
from jax.experimental import pallas as pl
from jax.experimental.pallas import tpu as pltpu
from jax.experimental.pallas import tpu_sc as plsc

NC, NS, LANES = 2, 16, 16
TILES = NC * NS
K = 80


def _edge_proj_body(ea_ref, w_ref, b_ref, o_ref):
    o_ref[...] = (
        jnp.dot(ea_ref[...], w_ref[...], preferred_element_type=jnp.float32)
        + b_ref[...]
    )


def _edge_proj(ea, w_t, b_row):
    Ep, DE = ea.shape
    D = w_t.shape[1]
    BE = TILES * K
    assert Ep % BE == 0
    return pl.pallas_call(
        _edge_proj_body,
        grid=(Ep // BE,),
        in_specs=[
            pl.BlockSpec((BE, DE), lambda i: (i, 0)),
            pl.BlockSpec((DE, D), lambda i: (0, 0)),
            pl.BlockSpec((1, D), lambda i: (0, 0)),
        ],
        out_specs=pl.BlockSpec((BE, D), lambda i: (i, 0)),
        out_shape=jax.ShapeDtypeStruct((Ep, D), jnp.float32),
        compiler_params=pltpu.CompilerParams(
            dimension_semantics=("parallel",)
        ),
    )(ea, w_t, b_row)


def _sc_agg(x, src, dst, e, zblk, n_chunks, n_pad):
    N, D = x.shape
    assert n_chunks % 2 == 0

    def body(x_hbm, src_hbm, dst_hbm, e_hbm, z_hbm, out_hbm,
             idx_s0, idx_d0, xj0, ev0, idx_s1, idx_d1, xj1, ev1,
             sg0, se0, sg1, sg1b, agg_sh):
        cid = lax.axis_index("c")
        sid = lax.axis_index("s")
        wid = sid * NC + cid
        zrows = n_pad // NS
        pltpu.sync_copy(z_hbm, agg_sh.at[pl.ds(sid * zrows, zrows)])
        plsc.subcore_barrier()

        base = wid * (n_chunks * K)
        bufs = ((idx_s0, idx_d0, xj0, ev0, sg0, se0),
                (idx_s1, idx_d1, xj1, ev1, sg1, sg1b))

        def start(j, b):
            idx_s, idx_d, xj, ev, sg, se = bufs[b]
            off = base + j * K
            pltpu.sync_copy(src_hbm.at[pl.ds(off, K)], idx_s)
            pltpu.sync_copy(dst_hbm.at[pl.ds(off, K)], idx_d)
            pltpu.make_async_copy(x_hbm.at[idx_s], xj, sg).start()
            pltpu.make_async_copy(e_hbm.at[pl.ds(off, K), :], ev, se).start()

        def finish(j, b):
            idx_s, idx_d, xj, ev, sg, se = bufs[b]
            off = base + j * K
            pltpu.make_async_copy(x_hbm.at[idx_s], xj, sg).wait()
            pltpu.make_async_copy(e_hbm.at[pl.ds(off, K), :], ev, se).wait()

            @plsc.parallel_loop(0, K, unroll=2)
            def _row(r):
                for cc in range(D // LANES):
                    col = pl.ds(cc * LANES, LANES)
                    xj[r, col] = jnp.maximum(xj[r, col] + ev[r, col], 0.0)

            pltpu.sync_copy(xj, agg_sh.at[idx_d], add=True)

        start(0, 0)

        @pl.loop(0, n_chunks, step=2)
        def _chunk(t):
            start(t + 1, 1)
            finish(t, 0)

            @pl.when(t + 2 < n_chunks)
            def _():
                start(t + 2, 0)

            finish(t + 1, 1)

        plsc.subcore_barrier()
        drows = ((N // NS) + 7) // 8 * 8
        last = N - (NS - 1) * drows

        @pl.when(sid < NS - 1)
        def _():
            pltpu.sync_copy(
                agg_sh.at[pl.ds(sid * drows, drows)],
                out_hbm.at[cid, pl.ds(sid * drows, drows), :],
            )

        @pl.when(sid == NS - 1)
        def _():
            pltpu.sync_copy(
                agg_sh.at[pl.ds((NS - 1) * drows, last)],
                out_hbm.at[cid, pl.ds((NS - 1) * drows, last), :],
            )

    mesh = plsc.VectorSubcoreMesh(core_axis_name="c", subcore_axis_name="s")
    kern = pl.kernel(
        body,
        out_type=jax.ShapeDtypeStruct((NC, N, D), jnp.float32),
        mesh=mesh,
        scratch_types=[
            pltpu.VMEM((K,), jnp.int32),
            pltpu.VMEM((K,), jnp.int32),
            pltpu.VMEM((K, D), jnp.float32),
            pltpu.VMEM((K, D), jnp.float32),
            pltpu.VMEM((K,), jnp.int32),
            pltpu.VMEM((K,), jnp.int32),
            pltpu.VMEM((K, D), jnp.float32),
            pltpu.VMEM((K, D), jnp.float32),
            pltpu.SemaphoreType.DMA,
            pltpu.SemaphoreType.DMA,
            pltpu.SemaphoreType.DMA,
            pltpu.SemaphoreType.DMA,
            pltpu.VMEM_SHARED((n_pad, D), jnp.float32),
        ],
    )
    return kern(x, src, dst, e, zblk)


def _node_mlp_body(x_ref, p_ref, w_ref, b_ref, o_ref):
    s = x_ref[...] + p_ref[0] + p_ref[1]
    h = jnp.dot(s, w_ref[...], preferred_element_type=jnp.float32) + b_ref[...]
    o_ref[...] = jnp.maximum(h, 0.0)


def _node_mlp(x, partials, w_t, b_row):
    N, D = x.shape
    BN = 1000
    assert N % BN == 0
    return pl.pallas_call(
        _node_mlp_body,
        grid=(N // BN,),
        in_specs=[
            pl.BlockSpec((BN, D), lambda i: (i, 0)),
            pl.BlockSpec((NC, BN, D), lambda i: (0, i, 0)),
            pl.BlockSpec((D, D), lambda i: (0, 0)),
            pl.BlockSpec((1, D), lambda i: (0, 0)),
        ],
        out_specs=pl.BlockSpec((BN, D), lambda i: (i, 0)),
        out_shape=jax.ShapeDtypeStruct((N, D), jnp.float32),
        compiler_params=pltpu.CompilerParams(
            dimension_semantics=("parallel",)
        ),
    )(x, partials, w_t, b_row)


def kernel(x, edge_index, edge_attr, W_edge, b_edge, W_mlp, b_mlp):
    N, D = x.shape
    E = edge_index.shape[1]
    DE = edge_attr.shape[1]

    per_round = TILES * K
    n_chunks = -(-E // per_round)
    n_chunks += n_chunks % 2
    Ep = per_round * n_chunks
    pad = Ep - E

    src = edge_index[0]
    dst = edge_index[1]
    ea = edge_attr
    if pad:
        src = jnp.concatenate([src, jnp.zeros((pad,), src.dtype)])
        dst = jnp.concatenate([dst, jnp.full((pad,), N, dst.dtype)])
        ea = jnp.concatenate([ea, jnp.zeros((pad, DE), ea.dtype)])

    n_pad = ((N + 1 + 8 * NS - 1) // (8 * NS)) * (8 * NS)
    zblk = jnp.zeros((n_pad // NS, D), jnp.float32)

    e = _edge_proj(ea, W_edge.T, b_edge[None, :])
    partials = _sc_agg(x, src, dst, e, zblk, n_chunks, n_pad)
    return _node_mlp(x, partials, W_mlp.T, b_mlp[None, :])
